# baseline (speedup 1.0000x reference)
import functools

import jax
import jax.numpy as jnp
from jax import lax
from jax.experimental import pallas as pl
from jax.experimental.pallas import tpu as pltpu
from jax.experimental.pallas import tpu_sc as plsc

B = 4096
L = 200
D = 128
NLANE = 16
NVREG = D // NLANE
LHALF = L // 2
LPAD = 104
NC, NS = 2, 16
NW = NC * NS
EPB = B // NW


NBUF = 2
UNROLL = 4


def _accumulate(buf, acc):
    def body(r, acc):
        for j in range(UNROLL):
            acc = tuple(acc[c] + buf[r * UNROLL + j, pl.ds(c * NLANE, NLANE)]
                        for c in range(NVREG))
        return acc

    return lax.fori_loop(0, LHALF // UNROLL, body, acc)


def _pool_one_table(tbl_hbm, idx3_hbm, out_hbm, idx_v, bufs, stage,
                    sems, base):
    pltpu.sync_copy(idx3_hbm.at[pl.ds(base, EPB)], idx_v)
    for b in range(NBUF):
        pltpu.async_copy(tbl_hbm.at[idx_v.at[b // 2, b % 2, pl.ds(0, LHALF)]],
                         bufs[b].at[pl.ds(0, LHALF)], sems[b])

    ngroups = 2 * EPB // NBUF

    def body(g, carry):
        for e in range(NBUF // 2):
            acc = tuple(jnp.zeros((NLANE,), jnp.float32) for _ in range(NVREG))
            for h in range(2):
                b = 2 * e + h
                pltpu.make_async_copy(
                    tbl_hbm.at[idx_v.at[0, 0, pl.ds(0, LHALF)]],
                    bufs[b].at[pl.ds(0, LHALF)], sems[b]).wait()
                acc = _accumulate(bufs[b], acc)

                @pl.when(g + 1 < ngroups)
                def _():
                    nxt = (NBUF // 2) * (g + 1) + e
                    pltpu.async_copy(
                        tbl_hbm.at[idx_v.at[nxt, h, pl.ds(0, LHALF)]],
                        bufs[b].at[pl.ds(0, LHALF)], sems[b])

            elem = (NBUF // 2) * g + e
            for c in range(NVREG):
                stage[elem, pl.ds(c * NLANE, NLANE)] = acc[c]
        return carry

    lax.fori_loop(0, ngroups, body, 0)
    pltpu.sync_copy(stage, out_hbm.at[pl.ds(base, EPB)])


def _pool_kernel(qs_hbm, ds_hbm, qt_hbm, dt_hbm, qo_hbm, do_hbm,
                 idx_v, *rest):
    wid = lax.axis_index("s") * NC + lax.axis_index("c")
    base = wid * EPB
    bufs = rest[:NBUF]
    stage = rest[NBUF]
    sems = rest[NBUF + 1:]
    _pool_one_table(qt_hbm, qs_hbm, qo_hbm, idx_v, bufs, stage, sems, base)
    _pool_one_table(dt_hbm, ds_hbm, do_hbm, idx_v, bufs, stage, sems, base)


@functools.cache
def _pool():
    return pl.kernel(
        _pool_kernel,
        out_type=(
            jax.ShapeDtypeStruct((B, D), jnp.float32),
            jax.ShapeDtypeStruct((B, D), jnp.float32),
        ),
        mesh=plsc.VectorSubcoreMesh(core_axis_name="c", subcore_axis_name="s"),
        scratch_types=(
            [pltpu.VMEM((EPB, 2, LPAD), jnp.int32)]
            + [pltpu.VMEM((LPAD, D), jnp.float32) for _ in range(NBUF)]
            + [pltpu.VMEM((EPB, D), jnp.float32)]
            + [pltpu.SemaphoreType.DMA for _ in range(NBUF)]
        ),
    )


def _tail_kernel(qsum_ref, dsum_ref, out_ref):
    scale = jnp.float32(1.0 / L)
    q = jnp.tanh(qsum_ref[...] * scale)
    d = jnp.tanh(dsum_ref[...] * scale)
    eps = jnp.float32(1e-12)
    nq = jnp.maximum(jnp.sqrt(jnp.sum(q * q, axis=1, keepdims=True)), eps)
    nd = jnp.maximum(jnp.sqrt(jnp.sum(d * d, axis=1, keepdims=True)), eps)
    num = jnp.sum(q * d, axis=1, keepdims=True)
    out_ref[...] = num / (nq * nd)


def kernel(qs, ds, q_table, d_table):
    padv = jnp.arange(B * 2 * (LPAD - LHALF), dtype=jnp.int32)
    padv = padv.reshape(B, 2, LPAD - LHALF) % jnp.int32(q_table.shape[0])
    qs3 = jnp.concatenate([qs.reshape(B, 2, LHALF), padv], axis=2)
    ds3 = jnp.concatenate([ds.reshape(B, 2, LHALF), padv], axis=2)
    q_sum, d_sum = _pool()(qs3, ds3, q_table, d_table)
    sims = pl.pallas_call(
        _tail_kernel,
        out_shape=jax.ShapeDtypeStruct((B, 1), jnp.float32),
    )(q_sum, d_sum)
    return sims.reshape(B)


# speedup vs baseline: 1.4767x; 1.4767x over previous
import functools

import jax
import jax.numpy as jnp
from jax import lax
from jax.experimental import pallas as pl
from jax.experimental.pallas import tpu as pltpu
from jax.experimental.pallas import tpu_sc as plsc

B = 4096
L = 200
D = 128
NLANE = 16
NVREG = D // NLANE
LHALF = L // 2
LPAD = 104
NC, NS = 2, 16
NW = NC * NS
EPB = B // NW


NBUF = 6
UNROLL = 4
NCHUNKS = 2 * EPB
NGROUPS = NCHUNKS // NBUF
NREM = (NCHUNKS - NGROUPS * NBUF) // 2


def _accumulate(buf, acc):
    def body(r, acc):
        for j in range(UNROLL):
            acc = tuple(acc[c] + buf[r * UNROLL + j, pl.ds(c * NLANE, NLANE)]
                        for c in range(NVREG))
        return acc

    return lax.fori_loop(0, LHALF // UNROLL, body, acc)


def _pool_one_table(tbl_hbm, idx3_hbm, out_hbm, idx_v, bufs, stage,
                    sems, base):
    pltpu.sync_copy(idx3_hbm.at[pl.ds(base, EPB)], idx_v)
    for b in range(NBUF):
        pltpu.async_copy(tbl_hbm.at[idx_v.at[b // 2, b % 2, pl.ds(0, LHALF)]],
                         bufs[b], sems[b])

    def consume(g, e, h, refill):
        b = 2 * e + h

        def go(acc):
            pltpu.make_async_copy(
                tbl_hbm.at[idx_v.at[0, 0, pl.ds(0, LHALF)]],
                bufs[b], sems[b]).wait()
            acc = _accumulate(bufs[b], acc)
            if refill:
                @pl.when(NBUF * (g + 1) + b < NCHUNKS)
                def _():
                    nxt = (NBUF // 2) * (g + 1) + e
                    pltpu.async_copy(
                        tbl_hbm.at[idx_v.at[nxt, h, pl.ds(0, LHALF)]],
                        bufs[b], sems[b])
            return acc

        return go

    def store(elem, acc):
        for c in range(NVREG):
            stage[elem, pl.ds(c * NLANE, NLANE)] = acc[c]

    def body(g, carry):
        for e in range(NBUF // 2):
            acc = tuple(jnp.zeros((NLANE,), jnp.float32) for _ in range(NVREG))
            for h in range(2):
                acc = consume(g, e, h, refill=True)(acc)
            store((NBUF // 2) * g + e, acc)
        return carry

    lax.fori_loop(0, NGROUPS, body, 0)
    for e in range(NREM):
        acc = tuple(jnp.zeros((NLANE,), jnp.float32) for _ in range(NVREG))
        for h in range(2):
            acc = consume(NGROUPS, e, h, refill=False)(acc)
        store((NBUF // 2) * NGROUPS + e, acc)
    pltpu.sync_copy(stage, out_hbm.at[pl.ds(base, EPB)])


def _pool_kernel(qs_hbm, ds_hbm, qt_hbm, dt_hbm, qo_hbm, do_hbm,
                 idx_v, *rest):
    wid = lax.axis_index("s") * NC + lax.axis_index("c")
    base = wid * EPB
    bufs = rest[:NBUF]
    stage = rest[NBUF]
    sems = rest[NBUF + 1:]
    _pool_one_table(qt_hbm, qs_hbm, qo_hbm, idx_v, bufs, stage, sems, base)
    _pool_one_table(dt_hbm, ds_hbm, do_hbm, idx_v, bufs, stage, sems, base)


@functools.cache
def _pool():
    return pl.kernel(
        _pool_kernel,
        out_type=(
            jax.ShapeDtypeStruct((B, D), jnp.float32),
            jax.ShapeDtypeStruct((B, D), jnp.float32),
        ),
        mesh=plsc.VectorSubcoreMesh(core_axis_name="c", subcore_axis_name="s"),
        scratch_types=(
            [pltpu.VMEM((EPB, 2, LPAD), jnp.int32)]
            + [pltpu.VMEM((LHALF, D), jnp.float32) for _ in range(NBUF)]
            + [pltpu.VMEM((EPB, D), jnp.float32)]
            + [pltpu.SemaphoreType.DMA for _ in range(NBUF)]
        ),
    )


def _tail_kernel(qsum_ref, dsum_ref, out_ref):
    scale = jnp.float32(1.0 / L)
    q = jnp.tanh(qsum_ref[...] * scale)
    d = jnp.tanh(dsum_ref[...] * scale)
    eps = jnp.float32(1e-12)
    nq = jnp.maximum(jnp.sqrt(jnp.sum(q * q, axis=1, keepdims=True)), eps)
    nd = jnp.maximum(jnp.sqrt(jnp.sum(d * d, axis=1, keepdims=True)), eps)
    num = jnp.sum(q * d, axis=1, keepdims=True)
    out_ref[...] = num / (nq * nd)


def kernel(qs, ds, q_table, d_table):
    padv = jnp.arange(B * 2 * (LPAD - LHALF), dtype=jnp.int32)
    padv = padv.reshape(B, 2, LPAD - LHALF) % jnp.int32(q_table.shape[0])
    qs3 = jnp.concatenate([qs.reshape(B, 2, LHALF), padv], axis=2)
    ds3 = jnp.concatenate([ds.reshape(B, 2, LHALF), padv], axis=2)
    q_sum, d_sum = _pool()(qs3, ds3, q_table, d_table)
    sims = pl.pallas_call(
        _tail_kernel,
        out_shape=jax.ShapeDtypeStruct((B, 1), jnp.float32),
    )(q_sum, d_sum)
    return sims.reshape(B)


# speedup vs baseline: 1.6148x; 1.0936x over previous
import functools

import jax
import jax.numpy as jnp
from jax import lax
from jax.experimental import pallas as pl
from jax.experimental.pallas import tpu as pltpu
from jax.experimental.pallas import tpu_sc as plsc

B = 4096
L = 200
D = 128
NLANE = 16
NVREG = D // NLANE
LHALF = L // 2
NC, NS = 2, 16
NW = NC * NS
EPB = B // NW


NBUF = 6
UNROLL = 4
NCHUNKS = 2 * EPB
NGROUPS = NCHUNKS // NBUF
NREM = (NCHUNKS - NGROUPS * NBUF) // 2


def _accumulate(buf, nrows, acc):
    def body(r, acc):
        for j in range(UNROLL):
            acc = tuple(acc[c] + buf[r * UNROLL + j, pl.ds(c * NLANE, NLANE)]
                        for c in range(NVREG))
        return acc

    return lax.fori_loop(0, nrows // UNROLL, body, acc)


def _pool_one_table(tbl_hbm, idx_hbm, out_hbm, idx_v, bufs, stage,
                    sems, base):
    pltpu.sync_copy(idx_hbm.at[pl.ds(base, EPB)], idx_v)

    def idx_slice(elem, h):
        return idx_v.at[elem, h]

    for b in range(NBUF):
        pltpu.async_copy(tbl_hbm.at[idx_slice(b // 2, b % 2)],
                         bufs[b], sems[b])

    def consume(g, e, h, refill):
        b = 2 * e + h

        def go(acc):
            pltpu.make_async_copy(
                tbl_hbm.at[idx_slice(0, h)],
                bufs[b], sems[b]).wait()
            acc = _accumulate(bufs[b], LHALF, acc)
            if refill:
                @pl.when(NBUF * (g + 1) + b < NCHUNKS)
                def _():
                    nxt = (NBUF // 2) * (g + 1) + e
                    pltpu.async_copy(
                        tbl_hbm.at[idx_slice(nxt, h)],
                        bufs[b], sems[b])
            return acc

        return go

    def store(elem, acc):
        for c in range(NVREG):
            stage[elem, pl.ds(c * NLANE, NLANE)] = acc[c]

    def body(g, carry):
        for e in range(NBUF // 2):
            acc = tuple(jnp.zeros((NLANE,), jnp.float32) for _ in range(NVREG))
            for h in range(2):
                acc = consume(g, e, h, refill=True)(acc)
            store((NBUF // 2) * g + e, acc)
        return carry

    lax.fori_loop(0, NGROUPS, body, 0)
    for e in range(NREM):
        acc = tuple(jnp.zeros((NLANE,), jnp.float32) for _ in range(NVREG))
        for h in range(2):
            acc = consume(NGROUPS, e, h, refill=False)(acc)
        store((NBUF // 2) * NGROUPS + e, acc)
    pltpu.sync_copy(stage, out_hbm.at[pl.ds(base, EPB)])


def _pool_kernel(qs_hbm, ds_hbm, qt_hbm, dt_hbm, qo_hbm, do_hbm,
                 idx_v, *rest):
    wid = lax.axis_index("s") * NC + lax.axis_index("c")
    base = wid * EPB
    bufs = rest[:NBUF]
    stage = rest[NBUF]
    sems = rest[NBUF + 1:]
    _pool_one_table(qt_hbm, qs_hbm, qo_hbm, idx_v, bufs, stage, sems, base)
    _pool_one_table(dt_hbm, ds_hbm, do_hbm, idx_v, bufs, stage, sems, base)


@functools.cache
def _pool():
    return pl.kernel(
        _pool_kernel,
        out_type=(
            jax.ShapeDtypeStruct((B, D), jnp.float32),
            jax.ShapeDtypeStruct((B, D), jnp.float32),
        ),
        mesh=plsc.VectorSubcoreMesh(core_axis_name="c", subcore_axis_name="s"),
        scratch_types=(
            [pltpu.VMEM((EPB, 2, LHALF), jnp.int32)]
            + [pltpu.VMEM((LHALF, D), jnp.float32) for _ in range(NBUF)]
            + [pltpu.VMEM((EPB, D), jnp.float32)]
            + [pltpu.SemaphoreType.DMA for _ in range(NBUF)]
        ),
    )


def _tail_kernel(qsum_ref, dsum_ref, out_ref):
    scale = jnp.float32(1.0 / L)
    q = jnp.tanh(qsum_ref[...] * scale)
    d = jnp.tanh(dsum_ref[...] * scale)
    eps = jnp.float32(1e-12)
    nq = jnp.maximum(jnp.sqrt(jnp.sum(q * q, axis=1, keepdims=True)), eps)
    nd = jnp.maximum(jnp.sqrt(jnp.sum(d * d, axis=1, keepdims=True)), eps)
    num = jnp.sum(q * d, axis=1, keepdims=True)
    out_ref[...] = num / (nq * nd)


def kernel(qs, ds, q_table, d_table):
    q_sum, d_sum = _pool()(qs.reshape(B, 2, LHALF), ds.reshape(B, 2, LHALF),
                           q_table, d_table)
    sims = pl.pallas_call(
        _tail_kernel,
        out_shape=jax.ShapeDtypeStruct((B, 1), jnp.float32),
    )(q_sum, d_sum)
    return sims.reshape(B)


# speedup vs baseline: 1.6357x; 1.0129x over previous
import functools

import jax
import jax.numpy as jnp
from jax import lax
from jax.experimental import pallas as pl
from jax.experimental.pallas import tpu as pltpu
from jax.experimental.pallas import tpu_sc as plsc

B = 4096
L = 200
D = 128
NLANE = 16
NVREG = D // NLANE
LHALF = L // 2
NC, NS = 2, 16
NW = NC * NS
EPB = B // NW

NBUF = 6
UNROLL = 4
NCHUNKS = 2 * EPB
NGROUPS = NCHUNKS // NBUF
NREM = (NCHUNKS - NGROUPS * NBUF) // 2


def _accumulate(buf, nrows, acc):
    def body(r, acc):
        for j in range(UNROLL):
            acc = tuple(acc[c] + buf[r * UNROLL + j, pl.ds(c * NLANE, NLANE)]
                        for c in range(NVREG))
        return acc

    return lax.fori_loop(0, nrows // UNROLL, body, acc)


def _tanh16(x):
    e = jnp.exp(x * jnp.float32(2.0))
    return jnp.float32(1.0) - jnp.float32(2.0) / (e + jnp.float32(1.0))


def _rsqrt16(x):
    i = plsc.bitcast(x, jnp.int32)
    y = plsc.bitcast(jnp.int32(0x5F3759DF) - (i >> 1), jnp.float32)
    for _ in range(3):
        y = y * (jnp.float32(1.5) - jnp.float32(0.5) * x * y * y)
    return y


def _lanesum16(x):
    idx = lax.iota(jnp.int32, NLANE)
    dnums = lax.GatherDimensionNumbers(
        offset_dims=(), collapsed_slice_dims=(0,), start_index_map=(0,))
    for sh in (8, 4, 2, 1):
        perm = (idx ^ sh)[:, None]
        x = x + lax.gather(x, perm, dnums, slice_sizes=(1,),
                           mode=lax.GatherScatterMode.PROMISE_IN_BOUNDS)
    return x


def _pool_one_table(tbl_hbm, idx_hbm, idx_v, bufs, sems, base, finalize):
    pltpu.sync_copy(idx_hbm.at[pl.ds(base, EPB)], idx_v)

    for b in range(NBUF):
        pltpu.async_copy(tbl_hbm.at[idx_v.at[b // 2, b % 2]], bufs[b], sems[b])

    def consume(g, e, h, refill):
        b = 2 * e + h

        def go(acc):
            pltpu.make_async_copy(
                tbl_hbm.at[idx_v.at[0, 0]], bufs[b], sems[b]).wait()
            acc = _accumulate(bufs[b], LHALF, acc)
            if refill:
                @pl.when(NBUF * (g + 1) + b < NCHUNKS)
                def _():
                    nxt = (NBUF // 2) * (g + 1) + e
                    pltpu.async_copy(
                        tbl_hbm.at[idx_v.at[nxt, h]], bufs[b], sems[b])
            return acc

        return go

    def body(g, carry):
        for e in range(NBUF // 2):
            acc = tuple(jnp.zeros((NLANE,), jnp.float32) for _ in range(NVREG))
            for h in range(2):
                acc = consume(g, e, h, refill=True)(acc)
            finalize((NBUF // 2) * g + e, acc)
        return carry

    lax.fori_loop(0, NGROUPS, body, 0)
    for e in range(NREM):
        acc = tuple(jnp.zeros((NLANE,), jnp.float32) for _ in range(NVREG))
        for h in range(2):
            acc = consume(NGROUPS, e, h, refill=False)(acc)
        finalize((NBUF // 2) * NGROUPS + e, acc)


def _pool_kernel(qs_hbm, ds_hbm, qt_hbm, dt_hbm, out_hbm, idx_v, *rest):
    wid = lax.axis_index("s") * NC + lax.axis_index("c")
    base = wid * EPB
    bufs = rest[:NBUF]
    stage = rest[NBUF]
    sims_v = rest[NBUF + 1]
    sems = rest[NBUF + 2:]

    def stash_q(elem, acc):
        for c in range(NVREG):
            stage[elem, pl.ds(c * NLANE, NLANE)] = acc[c]

    scale = jnp.float32(1.0 / L)
    lane0 = lax.iota(jnp.int32, NLANE) == 0

    def finish_elem(elem, acc):
        qt = tuple(_tanh16(stage[elem, pl.ds(c * NLANE, NLANE)] * scale)
                   for c in range(NVREG))
        dt = tuple(_tanh16(acc[c] * scale) for c in range(NVREG))
        nqv = sum((q * q for q in qt), jnp.zeros((NLANE,), jnp.float32))
        ndv = sum((d * d for d in dt), jnp.zeros((NLANE,), jnp.float32))
        numv = sum((q * d for q, d in zip(qt, dt)),
                   jnp.zeros((NLANE,), jnp.float32))
        tiny = jnp.float32(1e-24)
        denom = (jnp.maximum(_lanesum16(nqv), tiny)
                 * jnp.maximum(_lanesum16(ndv), tiny))
        simv = _lanesum16(numv) * _rsqrt16(denom)
        plsc.store_scatter(
            sims_v, [jnp.broadcast_to(elem, (NLANE,)).astype(jnp.int32)],
            simv, mask=lane0)

    _pool_one_table(qt_hbm, qs_hbm, idx_v, bufs, sems, base, stash_q)
    _pool_one_table(dt_hbm, ds_hbm, idx_v, bufs, sems, base, finish_elem)
    pltpu.sync_copy(sims_v, out_hbm.at[pl.ds(base, EPB)])


@functools.cache
def _pool():
    return pl.kernel(
        _pool_kernel,
        out_type=jax.ShapeDtypeStruct((B,), jnp.float32),
        mesh=plsc.VectorSubcoreMesh(core_axis_name="c", subcore_axis_name="s"),
        compiler_params=pltpu.CompilerParams(needs_layout_passes=False),
        scratch_types=(
            [pltpu.VMEM((EPB, 2, LHALF), jnp.int32)]
            + [pltpu.VMEM((LHALF, D), jnp.float32) for _ in range(NBUF)]
            + [pltpu.VMEM((EPB, D), jnp.float32)]
            + [pltpu.VMEM((EPB,), jnp.float32)]
            + [pltpu.SemaphoreType.DMA for _ in range(NBUF)]
        ),
    )


def kernel(qs, ds, q_table, d_table):
    return _pool()(qs.reshape(B, 2, LHALF), ds.reshape(B, 2, LHALF),
                   q_table, d_table)


# speedup vs baseline: 1.6610x; 1.0155x over previous
import functools

import jax
import jax.numpy as jnp
from jax import lax
from jax.experimental import pallas as pl
from jax.experimental.pallas import tpu as pltpu
from jax.experimental.pallas import tpu_sc as plsc

B = 4096
L = 200
D = 128
NLANE = 16
NVREG = D // NLANE
LSPLIT = (128, 72)
NC, NS = 2, 16
NW = NC * NS
EPB = B // NW

NBUF = 6
UNROLL = 4
NCHUNKS = 2 * EPB
NGROUPS = NCHUNKS // NBUF
NREM = (NCHUNKS - NGROUPS * NBUF) // 2


def _accumulate(buf, nrows, acc):
    def body(r, acc):
        for j in range(UNROLL):
            acc = tuple(acc[c] + buf[r * UNROLL + j, pl.ds(c * NLANE, NLANE)]
                        for c in range(NVREG))
        return acc

    return lax.fori_loop(0, nrows // UNROLL, body, acc)


def _tanh16(x):
    e = jnp.exp(x * jnp.float32(2.0))
    return jnp.float32(1.0) - jnp.float32(2.0) / (e + jnp.float32(1.0))


def _rsqrt16(x):
    i = plsc.bitcast(x, jnp.int32)
    y = plsc.bitcast(jnp.int32(0x5F3759DF) - (i >> 1), jnp.float32)
    for _ in range(3):
        y = y * (jnp.float32(1.5) - jnp.float32(0.5) * x * y * y)
    return y


def _lanesum16(x):
    idx = lax.iota(jnp.int32, NLANE)
    dnums = lax.GatherDimensionNumbers(
        offset_dims=(), collapsed_slice_dims=(0,), start_index_map=(0,))
    for sh in (8, 4, 2, 1):
        perm = (idx ^ sh)[:, None]
        x = x + lax.gather(x, perm, dnums, slice_sizes=(1,),
                           mode=lax.GatherScatterMode.PROMISE_IN_BOUNDS)
    return x


def _pool_one_table(tbl_hbm, idx_hbm, idx_v, bufs, sems, base, finalize):
    pltpu.sync_copy(idx_hbm.at[pl.ds(base, EPB)], idx_v)

    def idx_slice(elem, h):
        return idx_v.at[elem, pl.ds(0 if h == 0 else LSPLIT[0], LSPLIT[h])]

    for b in range(NBUF):
        pltpu.async_copy(tbl_hbm.at[idx_slice(b // 2, b % 2)],
                         bufs[b], sems[b])

    def consume(g, e, h, refill):
        b = 2 * e + h

        def go(acc):
            pltpu.make_async_copy(
                tbl_hbm.at[idx_slice(0, h)], bufs[b], sems[b]).wait()
            acc = _accumulate(bufs[b], LSPLIT[h], acc)
            if refill:
                @pl.when(NBUF * (g + 1) + b < NCHUNKS)
                def _():
                    nxt = (NBUF // 2) * (g + 1) + e
                    pltpu.async_copy(
                        tbl_hbm.at[idx_slice(nxt, h)], bufs[b], sems[b])
            return acc

        return go

    def body(g, carry):
        for e in range(NBUF // 2):
            acc = tuple(jnp.zeros((NLANE,), jnp.float32) for _ in range(NVREG))
            for h in range(2):
                acc = consume(g, e, h, refill=True)(acc)
            finalize((NBUF // 2) * g + e, acc)
        return carry

    lax.fori_loop(0, NGROUPS, body, 0)
    for e in range(NREM):
        acc = tuple(jnp.zeros((NLANE,), jnp.float32) for _ in range(NVREG))
        for h in range(2):
            acc = consume(NGROUPS, e, h, refill=False)(acc)
        finalize((NBUF // 2) * NGROUPS + e, acc)


def _pool_kernel(qs_hbm, ds_hbm, qt_hbm, dt_hbm, out_hbm, idx_v, *rest):
    wid = lax.axis_index("s") * NC + lax.axis_index("c")
    base = wid * EPB
    bufs = rest[:NBUF]
    stage = rest[NBUF]
    sims_v = rest[NBUF + 1]
    sems = rest[NBUF + 2:]

    def stash_q(elem, acc):
        for c in range(NVREG):
            stage[elem, pl.ds(c * NLANE, NLANE)] = acc[c]

    scale = jnp.float32(1.0 / L)
    lane0 = lax.iota(jnp.int32, NLANE) == 0

    def finish_elem(elem, acc):
        qt = tuple(_tanh16(stage[elem, pl.ds(c * NLANE, NLANE)] * scale)
                   for c in range(NVREG))
        dt = tuple(_tanh16(acc[c] * scale) for c in range(NVREG))
        nqv = sum((q * q for q in qt), jnp.zeros((NLANE,), jnp.float32))
        ndv = sum((d * d for d in dt), jnp.zeros((NLANE,), jnp.float32))
        numv = sum((q * d for q, d in zip(qt, dt)),
                   jnp.zeros((NLANE,), jnp.float32))
        tiny = jnp.float32(1e-24)
        denom = (jnp.maximum(_lanesum16(nqv), tiny)
                 * jnp.maximum(_lanesum16(ndv), tiny))
        simv = _lanesum16(numv) * _rsqrt16(denom)
        plsc.store_scatter(
            sims_v, [jnp.broadcast_to(elem, (NLANE,)).astype(jnp.int32)],
            simv, mask=lane0)

    _pool_one_table(qt_hbm, qs_hbm, idx_v, bufs, sems, base, stash_q)
    _pool_one_table(dt_hbm, ds_hbm, idx_v, bufs, sems, base, finish_elem)
    pltpu.sync_copy(sims_v, out_hbm.at[pl.ds(base, EPB)])


@functools.cache
def _pool():
    return pl.kernel(
        _pool_kernel,
        out_type=jax.ShapeDtypeStruct((B,), jnp.float32),
        mesh=plsc.VectorSubcoreMesh(core_axis_name="c", subcore_axis_name="s"),
        compiler_params=pltpu.CompilerParams(needs_layout_passes=False),
        scratch_types=(
            [pltpu.VMEM((EPB, L), jnp.int32)]
            + [pltpu.VMEM((LSPLIT[b % 2], D), jnp.float32)
               for b in range(NBUF)]
            + [pltpu.VMEM((EPB, D), jnp.float32)]
            + [pltpu.VMEM((EPB,), jnp.float32)]
            + [pltpu.SemaphoreType.DMA for _ in range(NBUF)]
        ),
    )


def kernel(qs, ds, q_table, d_table):
    return _pool()(qs, ds, q_table, d_table)
